# baseline (speedup 1.0000x reference)
import functools

import jax
import jax.numpy as jnp
from jax import lax
from jax.experimental import pallas as pl
from jax.experimental.pallas import tpu as pltpu
from jax.experimental.pallas import tpu_sc as plsc

VOCAB = 100000
HIDDEN = 128
MAX_POS = 2048
B, T = 4, 2048
N_ROWS = B * T

_CHUNK = 128


def _make_sc_kernel():
    info = plsc.get_sparse_core_info()
    nc, ns = info.num_cores, info.num_subcores
    nw = nc * ns
    rows_w = N_ROWS // nw
    n_chunks = rows_w // _CHUNK
    n_slices = T // (rows_w * nc)
    stage_rows = (n_slices * rows_w) // ns

    mesh = plsc.VectorSubcoreMesh(core_axis_name="c", subcore_axis_name="s")

    @functools.partial(
        pl.kernel,
        mesh=mesh,
        out_type=jax.ShapeDtypeStruct((B, T, HIDDEN), jnp.float32),
        scratch_types=[
            pltpu.VMEM((n_chunks, _CHUNK), jnp.int32),
            pltpu.VMEM((rows_w, HIDDEN), jnp.float32),
            pltpu.VMEM_SHARED((n_slices * rows_w, HIDDEN), jnp.float32),
        ]
        + [pltpu.SemaphoreType.DMA] * (1 + 3 * n_chunks),
    )
    def sc_kernel(
        ids_hbm, tok_hbm, pos_hbm, out_hbm, idx_v, tok_v, pos_sh, *sems
    ):
        sem_i = sems[0]
        sem_p = sems[1 : 1 + n_chunks]
        sem_g = sems[1 + n_chunks : 1 + 2 * n_chunks]
        sem_o = sems[1 + 2 * n_chunks :]

        cid = lax.axis_index("c")
        sid = lax.axis_index("s")
        wid = sid * nc + cid
        base = wid * rows_w
        b = base // T
        col = lax.rem(base, T)

        idx_cp = [
            pltpu.async_copy(
                ids_hbm.at[b, pl.ds(col + c * _CHUNK, _CHUNK)],
                idx_v.at[c],
                sem_i,
            )
            for c in range(n_chunks)
        ]

        j_stage = sid // n_slices
        part = lax.rem(sid, n_slices)
        src_row = (nc * j_stage + cid) * rows_w + part * stage_rows
        pltpu.sync_copy(
            pos_hbm.at[pl.ds(src_row, stage_rows)],
            pos_sh.at[pl.ds(j_stage * rows_w + part * stage_rows, stage_rows)],
        )
        plsc.subcore_barrier()

        j_need = lax.rem(sid, n_slices)
        p_cp = [
            pltpu.async_copy(
                pos_sh.at[pl.ds(j_need * rows_w + c * _CHUNK, _CHUNK)],
                tok_v.at[pl.ds(c * _CHUNK, _CHUNK)],
                sem_p[c],
            )
            for c in range(n_chunks)
        ]

        for cp in idx_cp:
            cp.wait()

        g_cp = []
        for c in range(n_chunks):
            p_cp[c].wait()
            g_cp.append(
                pltpu.async_copy(
                    tok_hbm.at[idx_v.at[c]],
                    tok_v.at[pl.ds(c * _CHUNK, _CHUNK)],
                    sem_g[c],
                    add=True,
                )
            )
        out_cp = []
        for c in range(n_chunks):
            g_cp[c].wait()
            out_cp.append(
                pltpu.async_copy(
                    tok_v.at[pl.ds(c * _CHUNK, _CHUNK)],
                    out_hbm.at[b, pl.ds(col + c * _CHUNK, _CHUNK)],
                    sem_o[c],
                )
            )
        for cp in out_cp:
            cp.wait()
        plsc.subcore_barrier()

    return sc_kernel


def kernel(input_ids, token_table, pos_table):
    return _make_sc_kernel()(
        input_ids.astype(jnp.int32), token_table, pos_table
    )


# speedup vs baseline: 1.0105x; 1.0105x over previous
import functools

import jax
import jax.numpy as jnp
from jax import lax
from jax.experimental import pallas as pl
from jax.experimental.pallas import tpu as pltpu
from jax.experimental.pallas import tpu_sc as plsc

VOCAB = 100000
HIDDEN = 128
MAX_POS = 2048
B, T = 4, 2048
N_ROWS = B * T

_CHUNK = 128


def _make_sc_kernel():
    info = plsc.get_sparse_core_info()
    nc, ns = info.num_cores, info.num_subcores
    nw = nc * ns
    rows_w = N_ROWS // nw
    n_chunks = rows_w // _CHUNK
    n_slices = T // (rows_w * nc)
    stage_rows = (n_slices * rows_w) // ns

    mesh = plsc.VectorSubcoreMesh(core_axis_name="c", subcore_axis_name="s")

    @functools.partial(
        pl.kernel,
        mesh=mesh,
        out_type=jax.ShapeDtypeStruct((B, T, HIDDEN), jnp.float32),
        scratch_types=[
            pltpu.VMEM((n_chunks, _CHUNK), jnp.int32),
            pltpu.VMEM((n_chunks, _CHUNK), jnp.int32),
            pltpu.VMEM((rows_w, HIDDEN), jnp.float32),
            pltpu.VMEM_SHARED((n_slices * rows_w, HIDDEN), jnp.float32),
        ]
        + [pltpu.SemaphoreType.DMA] * (1 + 3 * n_chunks),
    )
    def sc_kernel(
        ids_hbm, tok_hbm, pos_hbm, out_hbm, idx_v, pidx_v, tok_v, pos_sh, *sems
    ):
        sem_i = sems[0]
        sem_g = sems[1 : 1 + n_chunks]
        sem_a = sems[1 + n_chunks : 1 + 2 * n_chunks]
        sem_o = sems[1 + 2 * n_chunks :]

        cid = lax.axis_index("c")
        sid = lax.axis_index("s")
        wid = sid * nc + cid
        base = wid * rows_w
        b = base // T
        col = lax.rem(base, T)

        idx_cp = [
            pltpu.async_copy(
                ids_hbm.at[b, pl.ds(col + c * _CHUNK, _CHUNK)],
                idx_v.at[c],
                sem_i,
            )
            for c in range(n_chunks)
        ]

        j_need = lax.rem(sid, n_slices)
        for c in range(n_chunks):
            for k in range(_CHUNK // 16):
                pidx_v[c, pl.ds(k * 16, 16)] = lax.iota(jnp.int32, 16) + (
                    j_need * rows_w + c * _CHUNK + k * 16
                )

        j_stage = sid // n_slices
        part = lax.rem(sid, n_slices)
        src_row = (nc * j_stage + cid) * rows_w + part * stage_rows
        pltpu.sync_copy(
            pos_hbm.at[pl.ds(src_row, stage_rows)],
            pos_sh.at[pl.ds(j_stage * rows_w + part * stage_rows, stage_rows)],
        )

        for cp in idx_cp:
            cp.wait()
        g_cp = [
            pltpu.async_copy(
                tok_hbm.at[idx_v.at[c]],
                tok_v.at[pl.ds(c * _CHUNK, _CHUNK)],
                sem_g[c],
            )
            for c in range(n_chunks)
        ]

        plsc.subcore_barrier()

        a_cp = []
        for c in range(n_chunks):
            g_cp[c].wait()
            a_cp.append(
                pltpu.async_copy(
                    pos_sh.at[pidx_v.at[c]],
                    tok_v.at[pl.ds(c * _CHUNK, _CHUNK)],
                    sem_a[c],
                    add=True,
                )
            )
        out_cp = []
        for c in range(n_chunks):
            a_cp[c].wait()
            out_cp.append(
                pltpu.async_copy(
                    tok_v.at[pl.ds(c * _CHUNK, _CHUNK)],
                    out_hbm.at[b, pl.ds(col + c * _CHUNK, _CHUNK)],
                    sem_o[c],
                )
            )
        for cp in out_cp:
            cp.wait()
        plsc.subcore_barrier()

    return sc_kernel


def kernel(input_ids, token_table, pos_table):
    return _make_sc_kernel()(
        input_ids.astype(jnp.int32), token_table, pos_table
    )
